# baseline (speedup 1.0000x reference)
import jax
import jax.numpy as jnp
from jax import lax
from jax.experimental import pallas as pl
from jax.experimental.pallas import tpu as pltpu
from jax.experimental.pallas import tpu_sc as plsc

B, S, NCB, D = 4, 2048, 8, 1024
N = B * S
NC, NS = 2, 16
NW = NC * NS
T = 16
NCHT = N // T
NBUF = 4


def _embed_body(apay_hbm, tpay_hbm, na_hbm, text_hbm, audio_hbm, out_hbm,
                pay_v, tp_v, na_v, acc, g0, g1, g2, g3, tbuf,
                s0, s1, s2, s3, st, soa, sot):
    gbufs = (g0, g1, g2, g3)
    sems = (s0, s1, s2, s3)
    wid = lax.axis_index("s") * NC + lax.axis_index("c")

    pltpu.sync_copy(na_hbm, na_v)
    a = na_v[pl.ds(0, T)][0]
    nca = (a + T - 1) // T
    nct = (N - a + T - 1) // T
    da = nca - wid
    dt = nct - wid
    nA_w = jnp.where(da > 0, (da + NW - 1) // NW, 0)
    nT_w = jnp.where(dt > 0, (dt + NW - 1) // NW, 0)

    dummy_rows = out_hbm.at[pl.ds(0, T)]

    def chunk(k, carry):
        ci = wid + k * NW

        @pl.when(k < nT_w)
        def _():
            @pl.when(k > 0)
            def _():
                pltpu.make_async_copy(dummy_rows, tbuf, sot).wait()
            pltpu.sync_copy(tpay_hbm.at[ci], tp_v)
            pltpu.async_copy(text_hbm.at[tp_v.at[0]], tbuf, st)

        @pl.when(k < nA_w)
        def _():
            @pl.when(k > 0)
            def _():
                pltpu.make_async_copy(dummy_rows, acc, soa).wait()
            pltpu.sync_copy(apay_hbm.at[ci], pay_v)
            pend = {}
            for j in range(NBUF):
                pend[j] = pltpu.async_copy(
                    audio_hbm.at[pay_v.at[j]], gbufs[j], sems[j])
            for j in range(NCB):
                pend[j].wait()
                gb = gbufs[j % NBUF]
                if j == 0:
                    def init_row(t, _):
                        for kk in range(64):
                            off = kk * 16
                            acc[t, pl.ds(off, 16)] = gb[t, pl.ds(off, 16)]
                        return 0
                    lax.fori_loop(0, T, init_row, 0, unroll=False)
                else:
                    def add_row(t, _):
                        for kk in range(64):
                            off = kk * 16
                            plsc.addupdate(acc.at[t, pl.ds(off, 16)],
                                           gb[t, pl.ds(off, 16)])
                        return 0
                    lax.fori_loop(0, T, add_row, 0, unroll=False)
                nj = j + NBUF
                if nj < NCB:
                    pend[nj] = pltpu.async_copy(
                        audio_hbm.at[pay_v.at[nj]], gbufs[nj % NBUF],
                        sems[nj % NBUF])
            pltpu.async_copy(acc, out_hbm.at[pay_v.at[NCB]], soa)

        @pl.when(k < nT_w)
        def _():
            pltpu.make_async_copy(text_hbm.at[tp_v.at[0]], tbuf, st).wait()
            pltpu.async_copy(tbuf, out_hbm.at[tp_v.at[1]], sot)

        return carry

    lax.fori_loop(0, jnp.maximum(nA_w, nT_w), chunk, 0, unroll=False)

    @pl.when(nA_w > 0)
    def _():
        pltpu.make_async_copy(dummy_rows, acc, soa).wait()

    @pl.when(nT_w > 0)
    def _():
        pltpu.make_async_copy(dummy_rows, tbuf, sot).wait()


@jax.jit
def _sc_embed(apay, tpay, na, text_table, audio_table):
    mesh = plsc.VectorSubcoreMesh(core_axis_name="c", subcore_axis_name="s")
    run = pl.kernel(
        _embed_body,
        out_type=jax.ShapeDtypeStruct((N, D), jnp.float32),
        mesh=mesh,
        scratch_types=[
            pltpu.VMEM((NCB + 1, T), jnp.int32),
            pltpu.VMEM((2, T), jnp.int32),
            pltpu.VMEM((T,), jnp.int32),
            pltpu.VMEM((T, D), jnp.float32),
            pltpu.VMEM((T, D), jnp.float32),
            pltpu.VMEM((T, D), jnp.float32),
            pltpu.VMEM((T, D), jnp.float32),
            pltpu.VMEM((T, D), jnp.float32),
            pltpu.VMEM((T, D), jnp.float32),
            pltpu.SemaphoreType.DMA,
            pltpu.SemaphoreType.DMA,
            pltpu.SemaphoreType.DMA,
            pltpu.SemaphoreType.DMA,
            pltpu.SemaphoreType.DMA,
            pltpu.SemaphoreType.DMA,
            pltpu.SemaphoreType.DMA,
        ],
    )
    return run(apay, tpay, na, text_table, audio_table)


def kernel(input_ids, audio_mask, text_table, audio_table, offsets):
    ii32 = input_ids.astype(jnp.int32)
    m = audio_mask.reshape(N).astype(jnp.int32)
    shifted = (ii32 * audio_mask[:, None, :].astype(jnp.bool_).astype(jnp.int32)
               + offsets.reshape(1, -1, 1).astype(jnp.int32))
    shifted_tm = shifted.transpose(0, 2, 1).reshape(N, NCB)
    tid_raw = ii32[:, 0, :].reshape(N)
    tok = jnp.arange(N, dtype=jnp.int32)

    a_total = m.sum()
    posA = jnp.cumsum(m) - m
    posT = jnp.cumsum(1 - m) - (1 - m)
    ia = jnp.where(m == 1, posA, N)
    it = jnp.where(m == 0, posT, N)

    apay = jnp.zeros((N, NCB + 1), jnp.int32).at[ia].set(
        jnp.concatenate([shifted_tm, tok[:, None]], axis=1), mode="drop")
    tpay = jnp.zeros((N, 2), jnp.int32).at[it].set(
        jnp.stack([tid_raw, tok], axis=1), mode="drop")

    first = (tok // T) * T
    apay = jnp.where((tok >= a_total)[:, None], apay[first], apay)
    tpay = jnp.where((tok >= N - a_total)[:, None], tpay[first], tpay)

    apay = apay.reshape(NCHT, T, NCB + 1).transpose(0, 2, 1)
    tpay = tpay.reshape(NCHT, T, 2).transpose(0, 2, 1)
    na = jnp.full((T,), a_total, jnp.int32)

    out = _sc_embed(apay, tpay, na, text_table, audio_table)
    return out.reshape(B, S, D)


# speedup vs baseline: 1.1690x; 1.1690x over previous
import jax
import jax.numpy as jnp
from jax import lax
from jax.experimental import pallas as pl
from jax.experimental.pallas import tpu as pltpu
from jax.experimental.pallas import tpu_sc as plsc

B, S, NCB, D = 4, 2048, 8, 1024
N = B * S
NC, NS = 2, 16
NW = NC * NS
T = 16
NCH = N // T + 1
CPW = -(-NCH // NW)
NCHP = CPW * NW
NPOS = NCHP * T


def _embed_body(ids_hbm, dst_hbm, na_hbm, text_hbm, audio_hbm, out_hbm,
                ids_v, dst_v, na_v, g0, g1, g2, g3, g4, acc,
                s0, s1, s2, s3, s4, st, soa, sot):
    gb = (g0, g1, g2, g3, g4)
    gs = (s0, s1, s2, s3, s4)
    cid = lax.axis_index("c")
    sid = lax.axis_index("s")
    wid = sid * NC + cid

    pltpu.sync_copy(ids_hbm.at[wid], ids_v)
    pltpu.sync_copy(dst_hbm.at[wid], dst_v)
    pltpu.sync_copy(na_hbm, na_v)
    a = na_v[pl.ds(0, 16)][0]
    nca = (a + T - 1) // T
    nct = (N - a + T - 1) // T
    nctot = nca + nct
    dw = nctot - wid
    n_w = jnp.where(dw > 0, (dw + NW - 1) // NW, 0)

    dummy_rows = out_hbm.at[pl.ds(0, T)]

    def tree4(dest, b0, b1, b2, b3, accumulate):
        def row(t, _):
            for kk in range(64):
                off = kk * 16
                v = ((b0[t, pl.ds(off, 16)] + b1[t, pl.ds(off, 16)])
                     + (b2[t, pl.ds(off, 16)] + b3[t, pl.ds(off, 16)]))
                if accumulate:
                    plsc.addupdate(dest.at[t, pl.ds(off, 16)], v)
                else:
                    dest[t, pl.ds(off, 16)] = v
            return 0
        lax.fori_loop(0, T, row, 0, unroll=False)

    def chunk(k, carry):
        ci = wid + k * NW

        @pl.when(ci < nca)
        def _():
            pend = {}
            for j in range(5):
                pend[j] = pltpu.async_copy(
                    audio_hbm.at[ids_v.at[k, j]], gb[j], gs[j])
            for j in range(4):
                pend[j].wait()
            @pl.when(k > 0)
            def _():
                pltpu.make_async_copy(dummy_rows, acc, soa).wait()
            tree4(acc, g0, g1, g2, g3, False)
            for j in range(5, NCB):
                pend[j] = pltpu.async_copy(
                    audio_hbm.at[ids_v.at[k, j]], gb[j - 5], gs[j - 5])
            for j in range(4, NCB):
                pend[j].wait()
            tree4(acc, g4, g0, g1, g2, True)
            pltpu.async_copy(acc, out_hbm.at[dst_v.at[k]], soa)

        @pl.when(ci >= nca)
        def _():
            @pl.when((k > 0) & (ci - NW >= nca))
            def _():
                pltpu.make_async_copy(dummy_rows, g0, sot).wait()
            pltpu.async_copy(text_hbm.at[ids_v.at[k, 0]], g0, st).wait()
            pltpu.async_copy(g0, out_hbm.at[dst_v.at[k]], sot)

        return carry

    lax.fori_loop(0, n_w, chunk, 0, unroll=False)

    naw = jnp.where(nca > wid, (nca - wid + NW - 1) // NW, 0)

    @pl.when(naw > 0)
    def _():
        pltpu.make_async_copy(dummy_rows, acc, soa).wait()

    @pl.when(n_w > naw)
    def _():
        pltpu.make_async_copy(dummy_rows, g0, sot).wait()


@jax.jit
def _sc_embed(ids, dst, na, text_table, audio_table):
    mesh = plsc.VectorSubcoreMesh(core_axis_name="c", subcore_axis_name="s")
    run = pl.kernel(
        _embed_body,
        out_type=jax.ShapeDtypeStruct((N, D), jnp.float32),
        mesh=mesh,
        scratch_types=[
            pltpu.VMEM((CPW, NCB, T), jnp.int32),
            pltpu.VMEM((CPW, T), jnp.int32),
            pltpu.VMEM((16,), jnp.int32),
            pltpu.VMEM((T, D), jnp.float32),
            pltpu.VMEM((T, D), jnp.float32),
            pltpu.VMEM((T, D), jnp.float32),
            pltpu.VMEM((T, D), jnp.float32),
            pltpu.VMEM((T, D), jnp.float32),
            pltpu.VMEM((T, D), jnp.float32),
            pltpu.SemaphoreType.DMA,
            pltpu.SemaphoreType.DMA,
            pltpu.SemaphoreType.DMA,
            pltpu.SemaphoreType.DMA,
            pltpu.SemaphoreType.DMA,
            pltpu.SemaphoreType.DMA,
            pltpu.SemaphoreType.DMA,
            pltpu.SemaphoreType.DMA,
        ],
    )
    return run(ids, dst, na, text_table, audio_table)


def kernel(input_ids, audio_mask, text_table, audio_table, offsets):
    ii32 = input_ids.astype(jnp.int32)
    m = audio_mask.reshape(N).astype(jnp.int32)
    shifted = (ii32 * audio_mask[:, None, :].astype(jnp.bool_).astype(jnp.int32)
               + offsets.reshape(1, -1, 1).astype(jnp.int32))
    shifted_tm = shifted.transpose(0, 2, 1).reshape(N, NCB)
    tid_raw = ii32[:, 0, :].reshape(N)
    tok = jnp.arange(N, dtype=jnp.int32)

    a_total = m.sum()
    nca = (a_total + T - 1) // T
    text_start = nca * T
    posA = jnp.cumsum(m) - m
    posT = jnp.cumsum(1 - m) - (1 - m)
    pos = jnp.where(m == 1, posA, text_start + posT)

    ids8 = jnp.where((m == 1)[:, None], shifted_tm,
                     jnp.concatenate(
                         [tid_raw[:, None],
                          jnp.zeros((N, NCB - 1), jnp.int32)], axis=1))
    payload = jnp.concatenate([ids8, tok[:, None]], axis=1)
    P = jnp.zeros((NPOS, NCB + 1), jnp.int32).at[pos].set(payload)

    q = jnp.arange(NPOS, dtype=jnp.int32)
    is_real = ((q < a_total)
               | ((q >= text_start) & (q < text_start + (N - a_total))))
    Pr = P.reshape(NCHP, T, NCB + 1)
    Pr = jnp.where(is_real.reshape(NCHP, T)[:, :, None], Pr,
                   jnp.broadcast_to(Pr[:, 0:1, :], Pr.shape))

    Pw = Pr.reshape(CPW, NW, T, NCB + 1).transpose(1, 0, 3, 2)
    ids = Pw[:, :, :NCB, :]
    dst = Pw[:, :, NCB, :]
    na = jnp.full((16,), a_total, jnp.int32)

    out = _sc_embed(ids, dst, na, text_table, audio_table)
    return out.reshape(B, S, D)
